# baseline (speedup 1.0000x reference)
import functools

import jax
import jax.numpy as jnp
from jax import lax
from jax.experimental import pallas as pl
from jax.experimental.pallas import tpu as pltpu
from jax.experimental.pallas import tpu_sc as plsc

B = 16384
ND = 13
NS = 26
V = 100001
VP = 100096
ED = 16
H1, H2 = 64, 32

NCORES = 2
NSUB = 16
NW = NCORES * NSUB

NG = 2
NF = NS // NG
BCH = 512
NCHUNK = B // BCH
NTASK = NF * NCHUNK
TPW = NTASK // NW

ROWS_PER_FIELD = ED * (VP // 128)
SCRATCH_ROWS = NF * ROWS_PER_FIELD


WC = 4352
JC = WC // 8


def _detile_body(in_ref, out_ref):
    x = in_ref[0]
    pieces = [x[:, j * JC:(j + 1) * JC].T for j in range(8)]
    out_ref[...] = jnp.concatenate(pieces, axis=1)


def _tc_detile(table_t, lo):
    return pl.pallas_call(
        _detile_body,
        grid=(NF, VP // WC),
        in_specs=[pl.BlockSpec((1, ED, WC), lambda s, q: (s + lo, 0, q))],
        out_specs=pl.BlockSpec((JC, 128), lambda s, q: (s * (VP // WC) + q, 0)),
        out_shape=jax.ShapeDtypeStruct((NF * ED * VP // 128, 128), jnp.float32),
    )(table_t)


def _sc_gather_build():
    mesh = plsc.VectorSubcoreMesh(core_axis_name="c", subcore_axis_name="s")

    @functools.partial(
        pl.kernel,
        mesh=mesh,
        compiler_params=pltpu.CompilerParams(use_tc_tiling_on_sc=False),
        out_type=(
            jax.ShapeDtypeStruct((NCHUNK, BCH, NF, ED), jnp.float32),
            jax.ShapeDtypeStruct((NF, NCHUNK, BCH), jnp.float32),
        ),
        scratch_types=[
            pltpu.VMEM((2, BCH), jnp.int32),
            pltpu.VMEM((2, BCH), jnp.int32),
            pltpu.VMEM((2, BCH, ED), jnp.float32),
            pltpu.VMEM((2, BCH), jnp.float32),
            pltpu.SemaphoreType.DMA,
            pltpu.SemaphoreType.DMA,
        ],
    )
    def sc_gather(table_hbm, linw_hbm, idx_hbm, idxw_hbm, emb_out, wv_out,
                  idx_v, idxw_v, rows_v, wv_v, sem_g, sem_w):
        wid = lax.axis_index("s") * NCORES + lax.axis_index("c")
        for t in range(TPW):
            task = t * NW + wid
            b = t % 2
            s = task // NCHUNK
            c = task % NCHUNK
            boff = pl.multiple_of(c * BCH, BCH)
            pltpu.sync_copy(idx_hbm.at[s, pl.ds(boff, BCH)], idx_v.at[b])
            pltpu.sync_copy(idxw_hbm.at[s, pl.ds(boff, BCH)], idxw_v.at[b])
            cp_g = pltpu.async_copy(table_hbm.at[idx_v.at[b]],
                                    rows_v.at[b], sem_g)
            cpw = pltpu.async_copy(linw_hbm.at[0].at[idxw_v.at[b]],
                                   wv_v.at[b], sem_w)
            cp_g.wait()
            cpw.wait()
            pltpu.sync_copy(rows_v.at[b], emb_out.at[c, :, s, :])
            pltpu.sync_copy(wv_v.at[b], wv_out.at[s, c, :])

    return sc_gather


_sc_gather = _sc_gather_build()

def _wvsum_body(wva_ref, wvb_ref, out_ref):
    wva = wva_ref[...].reshape(NF, 128, 128).reshape(NF, B)
    wvb = wvb_ref[...].reshape(NF, 128, 128).reshape(NF, B)
    ones_col = jnp.ones((NF, 1), dtype=jnp.float32)
    out_ref[...] = (
        lax.dot_general(wva, ones_col, (((0,), (0,)), ((), ())),
                        preferred_element_type=jnp.float32)
        + lax.dot_general(wvb, ones_col, (((0,), (0,)), ((), ())),
                          preferred_element_type=jnp.float32)
    )


def _tc_wvsum(wva, wvb):
    return pl.pallas_call(
        _wvsum_body,
        out_shape=jax.ShapeDtypeStruct((B, 1), jnp.float32),
    )(wva.reshape(NF * 128, 128), wvb.reshape(NF * 128, 128))


BT = 2048


def _tc_mlp_body(xt_ref, emba_ref, embb_ref, wvs_ref,
                 w1_ref, b1_ref, w2_ref, b2_ref, wf_ref, bf_ref,
                 lww_ref, lwb_ref, out_ref):
    dense_t = xt_ref[:ND, :]
    emba = emba_ref[...]
    embb = embb_ref[...]
    wide = (
        lax.dot_general(dense_t, lww_ref[...], (((0,), (0,)), ((), ())),
                        preferred_element_type=jnp.float32)
        + lwb_ref[...]
        + wvs_ref[...]
    )
    h = lax.dot_general(dense_t, w1_ref[:ND, :], (((0,), (0,)), ((), ())),
                        preferred_element_type=jnp.float32)
    h += jnp.dot(emba, w1_ref[ND:ND + NF * ED, :],
                 preferred_element_type=jnp.float32)
    h += jnp.dot(embb, w1_ref[ND + NF * ED:, :],
                 preferred_element_type=jnp.float32)
    h = jax.nn.relu(h + b1_ref[...])
    h = jax.nn.relu(
        jnp.dot(h, w2_ref[...], preferred_element_type=jnp.float32) + b2_ref[...]
    )
    deep = jnp.dot(h, wf_ref[...], preferred_element_type=jnp.float32) + bf_ref[...]
    out_ref[...] = jax.nn.sigmoid(0.5 * wide + 0.5 * deep)


def _tc_mlp(xt, emba, embb, wvs, w1, b1r, w2, b2r, wf, bfr, lww, lwbr):
    rep = lambda shape: pl.BlockSpec(shape, lambda i: (0, 0))
    return pl.pallas_call(
        _tc_mlp_body,
        grid=(B // BT,),
        in_specs=[
            pl.BlockSpec((ND + NS, BT), lambda i: (0, i)),
            pl.BlockSpec((BT, NF * ED), lambda i: (i, 0)),
            pl.BlockSpec((BT, NF * ED), lambda i: (i, 0)),
            pl.BlockSpec((BT, 1), lambda i: (i, 0)),
            rep((ND + NS * ED, H1)),
            rep((1, H1)),
            rep((H1, H2)),
            rep((1, H2)),
            rep((H2, 1)),
            rep((1, 1)),
            rep((ND, 1)),
            rep((1, 1)),
        ],
        out_specs=pl.BlockSpec((BT, 1), lambda i: (i, 0)),
        out_shape=jax.ShapeDtypeStruct((B, 1), jnp.float32),
    )(xt, emba, embb, wvs, w1, b1r, w2, b2r, wf, bfr, lww, lwbr)


def kernel(inputs, embed_tables, linear_w, lw_W, lw_b, W1, b1, W2, b2, Wf, bf):
    table_t = jnp.transpose(embed_tables, (0, 2, 1))
    linw_t = jnp.transpose(linear_w, (1, 0))

    xt = jnp.transpose(inputs, (1, 0))
    idx_t = xt[ND:, :].astype(jnp.int32)
    idxw_t = idx_t + (jnp.arange(NS, dtype=jnp.int32) * V)[:, None]
    sloc = jnp.arange(NF, dtype=jnp.int32)[:, None]
    idx2_full = ((idx_t // WC) * WC + (idx_t % JC) * 8
                 + (idx_t % WC) // JC)

    halves = []
    for g in range(NG):
        lo = g * NF
        table_f = _tc_detile(table_t, lo).reshape(NF * VP, ED)
        e4, w3 = _sc_gather(table_f, linw_t,
                            idx2_full[lo:lo + NF] + sloc * VP,
                            idxw_t[lo:lo + NF])
        halves.append((e4.reshape(B, NF * ED), w3))
    (emba, wva), (embb, wvb) = halves

    wvs = _tc_wvsum(wva, wvb)
    out = _tc_mlp(
        xt,
        emba,
        embb,
        wvs,
        W1,
        b1.reshape(1, H1),
        W2,
        b2.reshape(1, H2),
        Wf,
        bf.reshape(1, 1),
        lw_W,
        lw_b.reshape(1, 1),
    )
    return out


# speedup vs baseline: 1.7970x; 1.7970x over previous
import functools

import jax
import jax.numpy as jnp
from jax import lax
from jax.experimental import pallas as pl
from jax.experimental.pallas import tpu as pltpu
from jax.experimental.pallas import tpu_sc as plsc

B = 16384
ND = 13
NS = 26
V = 100001
VP = 100096
ED = 16
H1, H2 = 64, 32

NCORES = 2
NSUB = 16
NW = NCORES * NSUB

NG = 2
NF = NS // NG
BCH = 1024
NCHUNK = B // BCH
NTASK = NF * NCHUNK
TPW = -(-NTASK // NW)

ROWS_PER_FIELD = ED * (VP // 128)
SCRATCH_ROWS = NF * ROWS_PER_FIELD


def _detile_body(in_ref, out_ref):
    out_ref[...] = in_ref[0].reshape(ROWS_PER_FIELD, 128)


def _tc_detile(table_t, lo):
    return pl.pallas_call(
        _detile_body,
        grid=(NF,),
        in_specs=[pl.BlockSpec((1, ED, VP), lambda s: (s + lo, 0, 0))],
        out_specs=pl.BlockSpec((ROWS_PER_FIELD, 128), lambda s: (s, 0)),
        out_shape=jax.ShapeDtypeStruct((SCRATCH_ROWS, 128), jnp.float32),
    )(table_t)


def _sc_gather_build():
    mesh = plsc.VectorSubcoreMesh(core_axis_name="c", subcore_axis_name="s")

    @functools.partial(
        pl.kernel,
        mesh=mesh,
        compiler_params=pltpu.CompilerParams(use_tc_tiling_on_sc=False),
        out_type=(
            jax.ShapeDtypeStruct((NF, ED, NCHUNK, BCH), jnp.float32),
            jax.ShapeDtypeStruct((NF, NCHUNK, BCH), jnp.float32),
        ),
        scratch_types=[
            pltpu.VMEM((2, BCH), jnp.int32),
            pltpu.VMEM((2, BCH), jnp.int32),
            pltpu.VMEM((2, ED, BCH), jnp.float32),
            pltpu.VMEM((2, BCH), jnp.float32),
            pltpu.SemaphoreType.DMA,
            pltpu.SemaphoreType.DMA,
            pltpu.SemaphoreType.DMA,
            pltpu.SemaphoreType.DMA,
        ],
    )
    def sc_gather(table_hbm, linw_hbm, idx_hbm, idxw_hbm, embt_out, wv_out,
                  idx_v, idxw_v, rows_v, wv_v, sem_i, sem_g, sem_w, sem_wb):
        wid = lax.axis_index("s") * NCORES + lax.axis_index("c")
        for t in range(TPW):
            task = t * NW + wid
            b = t % 2

            @pl.when(task < NTASK)
            def _(t=t, task=task, b=b):
                s = task // NCHUNK
                c = task % NCHUNK
                boff = pl.multiple_of(c * BCH, BCH)
                pltpu.sync_copy(idx_hbm.at[s, pl.ds(boff, BCH)], idx_v.at[b])
                pltpu.sync_copy(idxw_hbm.at[s, pl.ds(boff, BCH)],
                                idxw_v.at[b])
                cps = [
                    pltpu.async_copy(
                        table_hbm.at[pl.ds((s * ED + e) * VP, VP)]
                        .at[idx_v.at[b]],
                        rows_v.at[b, e], sem_g)
                    for e in range(ED)
                ]
                cpw = pltpu.async_copy(linw_hbm.at[0].at[idxw_v.at[b]],
                                       wv_v.at[b], sem_w)
                for cp in cps:
                    cp.wait()
                cpw.wait()
                pltpu.sync_copy(rows_v.at[b], embt_out.at[s, :, c, :])
                pltpu.sync_copy(wv_v.at[b], wv_out.at[s, c, :])

    return sc_gather


_sc_gather = _sc_gather_build()

def _tc_mlp_body(xt_ref, emba_ref, embb_ref, wva_ref, wvb_ref,
                 w1dt_ref, w1at_ref, w1bt_ref, b1_ref,
                 w2t_ref, b2_ref, wft_ref, bf_ref, lwwt_ref, lwb_ref, out_ref):
    dense_t = xt_ref[:ND, :]
    emba = emba_ref[...].reshape(NF * ED, 128, 128).reshape(NF * ED, B)
    embb = embb_ref[...].reshape(NF * ED, 128, 128).reshape(NF * ED, B)
    wva = wva_ref[...].reshape(NF, 128, 128).reshape(NF, B)
    wvb = wvb_ref[...].reshape(NF, 128, 128).reshape(NF, B)
    wide = (
        jnp.dot(lwwt_ref[...], dense_t, preferred_element_type=jnp.float32)
        + lwb_ref[...]
        + jnp.sum(wva, axis=0, keepdims=True)
        + jnp.sum(wvb, axis=0, keepdims=True)
    )
    h = jnp.dot(w1dt_ref[...], dense_t, preferred_element_type=jnp.float32)
    h += jnp.dot(w1at_ref[...], emba, preferred_element_type=jnp.float32)
    h += jnp.dot(w1bt_ref[...], embb, preferred_element_type=jnp.float32)
    h = jax.nn.relu(h + b1_ref[...])
    h = jax.nn.relu(
        jnp.dot(w2t_ref[...], h, preferred_element_type=jnp.float32) + b2_ref[...]
    )
    deep = jnp.dot(wft_ref[...], h, preferred_element_type=jnp.float32) + bf_ref[...]
    out_ref[...] = jax.nn.sigmoid(0.5 * wide + 0.5 * deep)


def _tc_mlp(xt, emba, embb, wva, wvb, w1dt, w1at, w1bt, b1c, w2t, b2c,
            wft, bfc, lwwt, lwbc):
    return pl.pallas_call(
        _tc_mlp_body,
        out_shape=jax.ShapeDtypeStruct((1, B), jnp.float32),
    )(xt, emba.reshape(NF * ED * 128, 128), embb.reshape(NF * ED * 128, 128),
      wva.reshape(NF * 128, 128), wvb.reshape(NF * 128, 128),
      w1dt, w1at, w1bt, b1c, w2t, b2c, wft, bfc, lwwt, lwbc)


def kernel(inputs, embed_tables, linear_w, lw_W, lw_b, W1, b1, W2, b2, Wf, bf):
    table_t = jnp.transpose(embed_tables, (0, 2, 1))
    linw_t = jnp.transpose(linear_w, (1, 0))

    xt = jnp.transpose(inputs, (1, 0))
    idx_t = xt[ND:, :].astype(jnp.int32)
    idxw_t = idx_t + (jnp.arange(NS, dtype=jnp.int32) * V)[:, None]

    halves = []
    for g in range(NG):
        lo = g * NF
        table_f = _tc_detile(table_t, lo).reshape(-1)
        e4, w3 = _sc_gather(table_f, linw_t,
                            idx_t[lo:lo + NF], idxw_t[lo:lo + NF])
        halves.append((e4.reshape(NF * ED, B), w3.reshape(NF, B)))
    (emba, wva), (embb, wvb) = halves

    out_row = _tc_mlp(
        xt,
        emba,
        embb,
        wva,
        wvb,
        W1[:ND].T,
        W1[ND:ND + NF * ED].T,
        W1[ND + NF * ED:].T,
        b1.reshape(H1, 1),
        W2.T,
        b2.reshape(H2, 1),
        Wf.T,
        bf.reshape(1, 1),
        lw_W.T,
        lw_b.reshape(1, 1),
    )
    return out_row.reshape(B, 1)
